# baseline (speedup 1.0000x reference)
import functools

import jax
import jax.numpy as jnp
from jax import lax
from jax.experimental import pallas as pl
from jax.experimental.pallas import tpu as pltpu


def _rot_c_body(j_ref, r_ref, wc1, bc1, wc2, bc2, out_ref):
    jr = j_ref[...] + r_ref[...]
    h = jnp.maximum(jr @ wc1[...] + bc1[...][None, :], 0.0)
    out_ref[...] = jnp.maximum(h @ wc2[...] + bc2[...][None, :], 0.0)


def _rot_p1_body(h_ref, wp1, bp1, out_ref):
    out_ref[...] = jnp.maximum(h_ref[...] @ wp1[...] + bp1[...][None, :], 0.0)


def _rot_tail_body(h_ref, wp2, bp2, wf1, bf1, wf2, bf2, wf3, bf3, wf4, bf4,
                   out_ref):
    h = jnp.maximum(h_ref[...] @ wp2[...] + bp2[...][None, :], 0.0)
    h = jnp.maximum(h @ wf1[...] + bf1[...][None, :], 0.0)
    h = jnp.maximum(h @ wf2[...] + bf2[...][None, :], 0.0)
    h = jnp.maximum(h @ wf3[...] + bf3[...][None, :], 0.0)
    out_ref[...] = h @ wf4[...] + bf4[...][None, :]


def _rot_branch(j_mid, r_mid, W_c1, b_c1, W_c2, b_c2, W_p1, b_p1, W_p2, b_p2,
                W_f1, b_f1, W_f2, b_f2, W_f3, b_f3, W_f4, b_f4):
    B = j_mid.shape[0]
    j = j_mid.reshape(B, 1024, 16).swapaxes(1, 2).reshape(B * 16, 1024)
    r = r_mid.reshape(B, 1024, 16).swapaxes(1, 2).reshape(B * 16, 1024)
    wp1p = W_p1.reshape(256, 16, 2048).transpose(1, 0, 2).reshape(4096, 2048)
    RT = 512
    full = lambda *s: pl.BlockSpec(s, lambda i: tuple(0 for _ in s))
    h2 = pl.pallas_call(
        _rot_c_body,
        grid=(B * 16 // RT,),
        in_specs=[
            pl.BlockSpec((RT, 1024), lambda i: (i, 0)),
            pl.BlockSpec((RT, 1024), lambda i: (i, 0)),
            full(1024, 512), full(512), full(512, 256), full(256),
        ],
        out_specs=pl.BlockSpec((RT, 256), lambda i: (i, 0)),
        out_shape=jax.ShapeDtypeStruct((B * 16, 256), jnp.float32),
    )(j, r, W_c1, b_c1, W_c2, b_c2).reshape(B, 4096)
    NT = 512
    h3 = pl.pallas_call(
        _rot_p1_body,
        grid=(2048 // NT,),
        in_specs=[
            pl.BlockSpec((B, 4096), lambda i: (0, 0)),
            pl.BlockSpec((4096, NT), lambda i: (0, i)),
            pl.BlockSpec((NT,), lambda i: (i,)),
        ],
        out_specs=pl.BlockSpec((B, NT), lambda i: (0, i)),
        out_shape=jax.ShapeDtypeStruct((B, 2048), jnp.float32),
    )(h2, wp1p, b_p1)
    return pl.pallas_call(
        _rot_tail_body,
        out_shape=jax.ShapeDtypeStruct((B, 6), jnp.float32),
    )(h3, W_p2, b_p2, W_f1, b_f1, W_f2, b_f2, W_f3, b_f3, W_f4, b_f4)


def _spiral_conv(xv, idx, W, b):
    g = jnp.take(xv, idx.reshape(-1), axis=1)
    g = g.reshape(xv.shape[0], idx.shape[0], -1)
    return jnp.einsum('bnf,fo->bno', g, W) + b


def _pool(xv, row, col, val):
    out = jnp.take(xv, col, axis=1) * val[None, :, None]
    n_out = row.shape[0] // 3
    base = jnp.zeros((xv.shape[0], n_out, xv.shape[-1]), dtype=xv.dtype)
    return base.at[:, row, :].add(out)


def _grid_sample(feat, uv):
    Bn, C, H, W = feat.shape
    xg = (uv[..., 0] + 1.0) * 0.5 * (W - 1)
    yg = (uv[..., 1] + 1.0) * 0.5 * (H - 1)
    x0 = jnp.floor(xg)
    y0 = jnp.floor(yg)
    wx1 = xg - x0
    wx0 = 1.0 - wx1
    wy1 = yg - y0
    wy0 = 1.0 - wy1
    x0i = jnp.clip(x0, 0, W - 1).astype(jnp.int32)
    x1i = jnp.clip(x0 + 1.0, 0, W - 1).astype(jnp.int32)
    y0i = jnp.clip(y0, 0, H - 1).astype(jnp.int32)
    y1i = jnp.clip(y0 + 1.0, 0, H - 1).astype(jnp.int32)
    gat = jax.vmap(lambda fb, yi, xi: fb[:, yi, xi])
    v00 = gat(feat, y0i, x0i)
    v01 = gat(feat, y0i, x1i)
    v10 = gat(feat, y1i, x0i)
    v11 = gat(feat, y1i, x1i)
    return (v00 * (wy0 * wx0)[:, None, :] + v01 * (wy0 * wx1)[:, None, :]
            + v10 * (wy1 * wx0)[:, None, :] + v11 * (wy1 * wx1)[:, None, :])


def _mesh_branch(uv, x, idxs, rows, cols, up_vals, W_dec, upsample,
                 de_params, W_head, b_head):
    uvc = jnp.clip((uv - 0.5) * 2.0, -1.0, 1.0)
    feat = jnp.einsum('bchw,cd->bdhw', x, W_dec)
    s = _grid_sample(feat, uvc)
    xm = jnp.transpose(s, (0, 2, 1))
    xm = jnp.einsum('vp,bpc->bvc', upsample, xm)
    for i in range(4):
        lvl = 3 - i
        xm = _pool(xm, rows[lvl], cols[lvl], up_vals[lvl])
        xm = jax.nn.relu(_spiral_conv(xm, idxs[lvl], de_params[i][0], de_params[i][1]))
    return _spiral_conv(xm, idxs[0], W_head, b_head)


def kernel(uv, x, j_mid, r_mid, spiral_idx_0, spiral_idx_1, spiral_idx_2, spiral_idx_3, up_row_0, up_col_0, up_val_0, up_row_1, up_col_1, up_val_1, up_row_2, up_col_2, up_val_2, up_row_3, up_col_3, up_val_3, W_dec, upsample, W_de_0, b_de_0, W_de_1, b_de_1, W_de_2, b_de_2, W_de_3, b_de_3, W_head, b_head, W_c1, b_c1, W_c2, b_c2, W_p1, b_p1, W_p2, b_p2, W_f1, b_f1, W_f2, b_f2, W_f3, b_f3, W_f4, b_f4):
    pred_rot = _rot_branch(j_mid, r_mid, W_c1, b_c1, W_c2, b_c2, W_p1, b_p1,
                           W_p2, b_p2, W_f1, b_f1, W_f2, b_f2, W_f3, b_f3,
                           W_f4, b_f4)
    pred = _mesh_branch(
        uv, x,
        (spiral_idx_0, spiral_idx_1, spiral_idx_2, spiral_idx_3),
        (up_row_0, up_row_1, up_row_2, up_row_3),
        (up_col_0, up_col_1, up_col_2, up_col_3),
        (up_val_0, up_val_1, up_val_2, up_val_3),
        W_dec, upsample,
        ((W_de_0, b_de_0), (W_de_1, b_de_1), (W_de_2, b_de_2), (W_de_3, b_de_3)),
        W_head, b_head)
    return (pred, pred_rot)
